# baseline (speedup 1.0000x reference)
import jax
import jax.numpy as jnp
from jax import lax
from jax.experimental import pallas as pl
from jax.experimental.pallas import tpu as pltpu

ROWS = 4096
FIELD_NUM = 26
EMBED_DIM = 64
COLS = FIELD_NUM * EMBED_DIM
NCAND = 128
BLOCK = 512


def _body(cand_ref, x_ref, o_ref, mask_ref):
    @pl.when(pl.program_id(0) == 0)
    def _():
        cols = lax.broadcasted_iota(jnp.int32, (8, COLS), 1)

        def step(j, m):
            return jnp.where(cols == cand_ref[j], 0.0, m)

        mask_ref[...] = lax.fori_loop(0, NCAND, step, jnp.ones((8, COLS), jnp.float32))

    o_ref[...] = x_ref[...] * mask_ref[0:1, :]


def kernel(embed, cand):
    x = embed.reshape(ROWS, COLS)
    out = pl.pallas_call(
        _body,
        grid=(ROWS // BLOCK,),
        in_specs=[
            pl.BlockSpec(memory_space=pltpu.SMEM),
            pl.BlockSpec((BLOCK, COLS), lambda i: (i, 0)),
        ],
        out_specs=pl.BlockSpec((BLOCK, COLS), lambda i: (i, 0)),
        out_shape=jax.ShapeDtypeStruct((ROWS, COLS), jnp.float32),
        scratch_shapes=[pltpu.VMEM((8, COLS), jnp.float32)],
    )(cand, x)
    return out.reshape(ROWS, FIELD_NUM, EMBED_DIM)
